# baseline (speedup 1.0000x reference)
import jax
import jax.numpy as jnp
from jax import lax
from jax.experimental import pallas as pl
from jax.experimental.pallas import tpu as pltpu
from jax.experimental.pallas import tpu_sc as plsc

N = 10000
E = 160000
D = 256
HD = 128
NS = 16
EPS = 10112
CH = 64
IR = 79
TAIL = 16
NW = 32
CHC = 40
NCC = (E // NW) // CHC
NZ = 10
RPS = N // NZ
NA = N + 16
ZR = 200
CW = 128

_f32 = jnp.float32


def _build_seg_sum():
    mesh = plsc.VectorSubcoreMesh(core_axis_name="c", subcore_axis_name="s")
    out_type = (jax.ShapeDtypeStruct((N, HD), _f32),
                jax.ShapeDtypeStruct((N, HD), _f32))
    scratch = [
        pltpu.VMEM((IR, 128), jnp.int32),
        pltpu.VMEM((IR, 128), jnp.int32),
        pltpu.VMEM((CH, HD), _f32),
        pltpu.VMEM((CH, HD), _f32),
        pltpu.VMEM_SHARED((NA, HD), _f32),
        pltpu.SemaphoreType.DMA,
        pltpu.SemaphoreType.DMA,
    ]

    def body(x0_hbm, x1_hbm, src_hbm, dst_hbm, z_hbm,
             s0_hbm, s1_hbm, src_v, dst_v, gbuf0, gbuf1, acc, sem0, sem1):
        cid = lax.axis_index("c")
        sid = lax.axis_index("s")
        base = sid * RPS

        @pl.when(sid < NZ)
        def _():
            pltpu.sync_copy(z_hbm, acc.at[pl.ds(base, RPS)])

        pltpu.sync_copy(src_hbm.at[sid], src_v)
        pltpu.sync_copy(dst_hbm.at[sid], dst_v)

        plsc.subcore_barrier()

        def run(x_hbm):
            lo = pl.ds(0, CH)
            hi = pl.ds(CH, CH)
            pltpu.async_copy(x_hbm.at[src_v.at[0, lo]], gbuf0, sem0)

            @pl.loop(0, IR - 1)
            def _(r):
                pltpu.make_async_copy(x_hbm.at[src_v.at[r, lo]], gbuf0,
                                      sem0).wait()
                pltpu.async_copy(x_hbm.at[src_v.at[r, hi]], gbuf1, sem1)
                pltpu.sync_copy(gbuf0, acc.at[dst_v.at[r, lo]], add=True)
                pltpu.make_async_copy(x_hbm.at[src_v.at[r, hi]], gbuf1,
                                      sem1).wait()

                @pl.when(r + 1 < IR - 1)
                def _():
                    pltpu.async_copy(x_hbm.at[src_v.at[r + 1, lo]], gbuf0,
                                     sem0)

                pltpu.sync_copy(gbuf1, acc.at[dst_v.at[r, hi]], add=True)

            tl = pl.ds(0, TAIL)
            pltpu.sync_copy(x_hbm.at[src_v.at[IR - 1, tl]], gbuf0.at[tl])
            pltpu.sync_copy(gbuf0.at[tl], acc.at[dst_v.at[IR - 1, tl]],
                            add=True)

        @pl.when(cid == 0)
        def _():
            run(x0_hbm)

        @pl.when(cid == 1)
        def _():
            run(x1_hbm)

        plsc.subcore_barrier()

        @pl.when((cid == 0) & (sid < NZ))
        def _():
            pltpu.sync_copy(acc.at[pl.ds(base, RPS)], s0_hbm.at[pl.ds(base, RPS)])

        @pl.when((cid == 1) & (sid < NZ))
        def _():
            pltpu.sync_copy(acc.at[pl.ds(base, RPS)], s1_hbm.at[pl.ds(base, RPS)])

    return pl.kernel(body, out_type=out_type, mesh=mesh,
                     scratch_types=scratch)


def _build_counts():
    mesh = plsc.VectorSubcoreMesh(core_axis_name="c", subcore_axis_name="s")
    out_type = (jax.ShapeDtypeStruct((N, CW), _f32),
                jax.ShapeDtypeStruct((N, CW), _f32))
    scratch = [
        pltpu.VMEM((NCC, CHC), jnp.int32),
        pltpu.VMEM((CHC, CW), _f32),
        pltpu.VMEM_SHARED((N, CW), _f32),
    ]

    def body(dst_hbm, ones_hbm, zc_hbm, c0_hbm, c1_hbm, dst_v, ones_v, cacc):
        cid = lax.axis_index("c")
        sid = lax.axis_index("s")
        wid = cid * NS + sid
        base = sid * RPS

        pltpu.sync_copy(ones_hbm, ones_v)
        pltpu.sync_copy(dst_hbm.at[wid], dst_v)

        @pl.when(sid < NZ)
        def _():
            pltpu.sync_copy(zc_hbm, cacc.at[pl.ds(base, RPS)])

        plsc.subcore_barrier()

        @pl.loop(0, NCC)
        def _(j):
            pltpu.sync_copy(ones_v, cacc.at[dst_v.at[j]], add=True)

        plsc.subcore_barrier()

        @pl.when((cid == 0) & (sid < NZ))
        def _():
            pltpu.sync_copy(cacc.at[pl.ds(base, RPS)], c0_hbm.at[pl.ds(base, RPS)])

        @pl.when((cid == 1) & (sid < NZ))
        def _():
            pltpu.sync_copy(cacc.at[pl.ds(base, RPS)], c1_hbm.at[pl.ds(base, RPS)])

    return pl.kernel(body, out_type=out_type, mesh=mesh,
                     scratch_types=scratch)


_R = 1000


def _embed(x, wt, b):
    def body(x_ref, wt_ref, b_ref, o0_ref, o1_ref):
        y = jnp.dot(x_ref[...], wt_ref[...], preferred_element_type=_f32)
        y = jnp.maximum(y + b_ref[...], 0.0)
        o0_ref[...] = y[:, :HD]
        o1_ref[...] = y[:, HD:]

    return pl.pallas_call(
        body,
        grid=(N // _R,),
        in_specs=[pl.BlockSpec((_R, D), lambda i: (i, 0)),
                  pl.BlockSpec((D, D), lambda i: (0, 0)),
                  pl.BlockSpec((1, D), lambda i: (0, 0))],
        out_specs=[pl.BlockSpec((_R, HD), lambda i: (i, 0)),
                   pl.BlockSpec((_R, HD), lambda i: (i, 0))],
        out_shape=[jax.ShapeDtypeStruct((N, HD), _f32),
                   jax.ShapeDtypeStruct((N, HD), _f32)],
    )(x, wt, b)


def _conv_combine(m1, c1, m2, c2, xs, wlt1, wlt2, wrts, bsum):
    def body(m10, m11, c1a, c1b, m20, m21, c2a, c2b, x0, x1, w1, w2, wr, br,
             o0_ref, o1_ref):
        inv1 = 1.0 / jnp.maximum(c1a[...][:, 0:1] + c1b[...][:, 0:1], 1.0)
        inv2 = 1.0 / jnp.maximum(c2a[...][:, 0:1] + c2b[...][:, 0:1], 1.0)
        mm1 = jnp.concatenate([m10[...], m11[...]], axis=1) * inv1
        mm2 = jnp.concatenate([m20[...], m21[...]], axis=1) * inv2
        xx = jnp.concatenate([x0[...], x1[...]], axis=1)
        acc = jnp.dot(mm1, w1[...], preferred_element_type=_f32)
        acc = acc + jnp.dot(mm2, w2[...], preferred_element_type=_f32)
        acc = acc + jnp.dot(xx, wr[...], preferred_element_type=_f32)
        y = jnp.maximum(acc + br[...], 0.0)
        o0_ref[...] = y[:, :HD]
        o1_ref[...] = y[:, HD:]

    half = pl.BlockSpec((_R, HD), lambda i: (i, 0))
    cspec = pl.BlockSpec((_R, CW), lambda i: (i, 0))
    wspec = pl.BlockSpec((D, D), lambda i: (0, 0))
    return pl.pallas_call(
        body,
        grid=(N // _R,),
        in_specs=[half, half, cspec, cspec, half, half, cspec, cspec,
                  half, half,
                  wspec, wspec, wspec, pl.BlockSpec((1, D), lambda i: (0, 0))],
        out_specs=[half, half],
        out_shape=[jax.ShapeDtypeStruct((N, HD), _f32),
                   jax.ShapeDtypeStruct((N, HD), _f32)],
    )(m1[0], m1[1], c1[0], c1[1], m2[0], m2[1], c2[0], c2[1], xs[0], xs[1],
      wlt1, wlt2, wrts, bsum)


def _rule_heads(xs, w1t, b1, w2t, b2, hwt, hb):
    def body(x0, x1, w1, b1r, w2, b2r, hw, hbr, o_ref):
        x = jnp.concatenate([x0[...], x1[...]], axis=1)
        h = jnp.dot(x, w1[...], preferred_element_type=_f32) + b1r[...]
        h = jnp.maximum(h, 0.0)
        y = x + 0.1 * (jnp.dot(h, w2[...], preferred_element_type=_f32)
                       + b2r[...])
        z = jnp.dot(y, hw[...], preferred_element_type=_f32) + hbr[...]
        o_ref[...] = jax.nn.sigmoid(z)

    half = pl.BlockSpec((_R, HD), lambda i: (i, 0))
    wspec = pl.BlockSpec((D, D), lambda i: (0, 0))
    bspec = pl.BlockSpec((1, D), lambda i: (0, 0))
    return pl.pallas_call(
        body,
        grid=(N // _R,),
        in_specs=[half, half, wspec, bspec, wspec, bspec,
                  pl.BlockSpec((D, 2), lambda i: (0, 0)),
                  pl.BlockSpec((1, 2), lambda i: (0, 0))],
        out_specs=pl.BlockSpec((_R, 2), lambda i: (i, 0)),
        out_shape=jax.ShapeDtypeStruct((N, 2), _f32),
    )(xs[0], xs[1], w1t, b1, w2t, b2, hwt, hb)


def _prep_edges(e):
    pad = EPS - E // NS
    src = jnp.pad(e[0].reshape(NS, E // NS), ((0, 0), (0, pad)),
                  constant_values=0)
    dst = jnp.pad(e[1].reshape(NS, E // NS), ((0, 0), (0, pad)),
                  constant_values=N)
    return (src.reshape(NS, IR, 128), dst.reshape(NS, IR, 128))


def _prep_cnt_edges(e):
    return e[1].reshape(NW, NCC, CHC)


def kernel(x_agent, x_track, edge_observes, edge_observed_by, edge_in_fov,
           edge_in_fov_by, params):
    p = params
    seg = _build_seg_sum()
    cnt_k = _build_counts()

    def lt(q):
        return q["W"].T, q["b"][None, :]

    wat, ba = lt(p["agent_emb"])
    wtt, bt = lt(p["track_emb"])

    def conv_w(cp, et1, et2):
        wlt1 = cp[et1]["Wl"].T
        wlt2 = cp[et2]["Wl"].T
        wrts = (cp[et1]["Wr"] + cp[et2]["Wr"]).T
        bsum = (cp[et1]["bl"] + cp[et2]["bl"])[None, :]
        return wlt1, wlt2, wrts, bsum

    a1w = conv_w(p["conv1"], "observed_by", "in_fov_by")
    t1w = conv_w(p["conv1"], "observes", "in_fov")
    a2w = conv_w(p["conv2"], "observed_by", "in_fov_by")
    t2w = conv_w(p["conv2"], "observes", "in_fov")

    r = p["rule"]
    w1t, b1 = r["W1"].T, r["b1"][None, :]
    w2t, b2 = r["W2"].T, r["b2"][None, :]
    h = p["heads"]
    a_hwt = jnp.concatenate([h["agent_value"]["W"].T, h["agent_conf"]["W"].T],
                            axis=1)
    a_hb = jnp.concatenate([h["agent_value"]["b"], h["agent_conf"]["b"]])[None, :]
    t_hwt = jnp.concatenate([h["track_value"]["W"].T, h["track_conf"]["W"].T],
                            axis=1)
    t_hb = jnp.concatenate([h["track_value"]["b"], h["track_conf"]["b"]])[None, :]

    zs = jnp.zeros((RPS, HD), _f32)
    zc = jnp.zeros((RPS, CW), _f32)
    ones = jnp.ones((CHC, CW), _f32)

    e_ob = _prep_edges(edge_observed_by) + (zs,)
    e_ifb = _prep_edges(edge_in_fov_by) + (zs,)
    e_obs = _prep_edges(edge_observes) + (zs,)
    e_if = _prep_edges(edge_in_fov) + (zs,)

    xa = _embed(x_agent, wat, ba)
    xt = _embed(x_track, wtt, bt)

    c_ob = cnt_k(_prep_cnt_edges(edge_observed_by), ones, zc)
    c_ifb = cnt_k(_prep_cnt_edges(edge_in_fov_by), ones, zc)
    c_obs = cnt_k(_prep_cnt_edges(edge_observes), ones, zc)
    c_if = cnt_k(_prep_cnt_edges(edge_in_fov), ones, zc)

    s_ob0, s_ob1 = seg(xt[0], xt[1], *e_ob)
    s_ifb0, s_ifb1 = seg(xt[0], xt[1], *e_ifb)
    s_obs0, s_obs1 = seg(xa[0], xa[1], *e_obs)
    s_if0, s_if1 = seg(xa[0], xa[1], *e_if)
    xa1 = _conv_combine((s_ob0, s_ob1), c_ob, (s_ifb0, s_ifb1), c_ifb,
                        xa, *a1w)
    xt1 = _conv_combine((s_obs0, s_obs1), c_obs, (s_if0, s_if1), c_if,
                        xt, *t1w)

    s_ob0, s_ob1 = seg(xt1[0], xt1[1], *e_ob)
    s_ifb0, s_ifb1 = seg(xt1[0], xt1[1], *e_ifb)
    s_obs0, s_obs1 = seg(xa1[0], xa1[1], *e_obs)
    s_if0, s_if1 = seg(xa1[0], xa1[1], *e_if)
    xa2 = _conv_combine((s_ob0, s_ob1), c_ob, (s_ifb0, s_ifb1), c_ifb,
                        xa1, *a2w)
    xt2 = _conv_combine((s_obs0, s_obs1), c_obs, (s_if0, s_if1), c_if,
                        xt1, *t2w)

    za = _rule_heads(xa2, w1t, b1, w2t, b2, a_hwt, a_hb)
    zt = _rule_heads(xt2, w1t, b1, w2t, b2, t_hwt, t_hb)

    av, ac = za[:, 0:1], za[:, 1:2]
    tv, tc = zt[:, 0:1], zt[:, 1:2]
    return av, ac, tv, tc


# speedup vs baseline: 1.0171x; 1.0171x over previous
import jax
import jax.numpy as jnp
from jax import lax
from jax.experimental import pallas as pl
from jax.experimental.pallas import tpu as pltpu
from jax.experimental.pallas import tpu_sc as plsc

N = 10000
E = 160000
D = 256
HD = 128
NS = 16
EPS = 10112
CH = 128
IR = 79
TAIL = 16
NW = 32
CRC = 40
TAILC = 8
NZ = 10
RPS = N // NZ
NA = N + 16
ZR = 200
CW = 128

_f32 = jnp.float32


def _build_seg_sum():
    mesh = plsc.VectorSubcoreMesh(core_axis_name="c", subcore_axis_name="s")
    out_type = (jax.ShapeDtypeStruct((N, HD), _f32),
                jax.ShapeDtypeStruct((N, HD), _f32))
    scratch = [
        pltpu.VMEM((IR, 128), jnp.int32),
        pltpu.VMEM((IR, 128), jnp.int32),
        pltpu.VMEM((CH, HD), _f32),
        pltpu.VMEM_SHARED((NA, HD), _f32),
        pltpu.SemaphoreType.DMA,
    ]

    def body(x0_hbm, x1_hbm, src_hbm, dst_hbm, z_hbm,
             s0_hbm, s1_hbm, src_v, dst_v, gbuf, acc, sem0):
        cid = lax.axis_index("c")
        sid = lax.axis_index("s")
        base = sid * RPS

        @pl.when(sid < NZ)
        def _():
            pltpu.sync_copy(z_hbm, acc.at[pl.ds(base, RPS)])

        pltpu.sync_copy(src_hbm.at[sid], src_v)
        pltpu.sync_copy(dst_hbm.at[sid], dst_v)

        plsc.subcore_barrier()

        def run(x_hbm):
            @pl.loop(0, IR - 1)
            def _(r):
                pltpu.sync_copy(x_hbm.at[src_v.at[r]], gbuf)
                pltpu.sync_copy(gbuf, acc.at[dst_v.at[r]], add=True)

            tl = pl.ds(0, TAIL)
            pltpu.sync_copy(x_hbm.at[src_v.at[IR - 1, tl]], gbuf.at[tl])
            pltpu.sync_copy(gbuf.at[tl], acc.at[dst_v.at[IR - 1, tl]],
                            add=True)

        @pl.when(cid == 0)
        def _():
            run(x0_hbm)

        @pl.when(cid == 1)
        def _():
            run(x1_hbm)

        plsc.subcore_barrier()

        @pl.when((cid == 0) & (sid < NZ))
        def _():
            pltpu.sync_copy(acc.at[pl.ds(base, RPS)], s0_hbm.at[pl.ds(base, RPS)])

        @pl.when((cid == 1) & (sid < NZ))
        def _():
            pltpu.sync_copy(acc.at[pl.ds(base, RPS)], s1_hbm.at[pl.ds(base, RPS)])

    return pl.kernel(body, out_type=out_type, mesh=mesh,
                     scratch_types=scratch)


def _build_counts():
    mesh = plsc.VectorSubcoreMesh(core_axis_name="c", subcore_axis_name="s")
    out_type = (jax.ShapeDtypeStruct((N, CW), _f32),
                jax.ShapeDtypeStruct((N, CW), _f32))
    scratch = [
        pltpu.VMEM((CRC, 128), jnp.int32),
        pltpu.VMEM((128, CW), _f32),
        pltpu.VMEM_SHARED((N, CW), _f32),
    ]

    def body(dst_hbm, ones_hbm, zc_hbm, c0_hbm, c1_hbm, dst_v, ones_v, cacc):
        cid = lax.axis_index("c")
        sid = lax.axis_index("s")
        wid = cid * NS + sid
        base = sid * RPS

        pltpu.sync_copy(ones_hbm, ones_v)
        pltpu.sync_copy(dst_hbm.at[wid], dst_v)

        @pl.when(sid < NZ)
        def _():
            pltpu.sync_copy(zc_hbm, cacc.at[pl.ds(base, RPS)])

        plsc.subcore_barrier()

        @pl.loop(0, CRC - 1)
        def _(j):
            pltpu.sync_copy(ones_v, cacc.at[dst_v.at[j]], add=True)

        tl = pl.ds(0, TAILC)
        pltpu.sync_copy(ones_v.at[tl], cacc.at[dst_v.at[CRC - 1, tl]],
                        add=True)

        plsc.subcore_barrier()

        @pl.when((cid == 0) & (sid < NZ))
        def _():
            pltpu.sync_copy(cacc.at[pl.ds(base, RPS)], c0_hbm.at[pl.ds(base, RPS)])

        @pl.when((cid == 1) & (sid < NZ))
        def _():
            pltpu.sync_copy(cacc.at[pl.ds(base, RPS)], c1_hbm.at[pl.ds(base, RPS)])

    return pl.kernel(body, out_type=out_type, mesh=mesh,
                     scratch_types=scratch)


_R = 1000


def _embed(x, wt, b):
    def body(x_ref, wt_ref, b_ref, o0_ref, o1_ref):
        y = jnp.dot(x_ref[...], wt_ref[...], preferred_element_type=_f32)
        y = jnp.maximum(y + b_ref[...], 0.0)
        o0_ref[...] = y[:, :HD]
        o1_ref[...] = y[:, HD:]

    return pl.pallas_call(
        body,
        grid=(N // _R,),
        in_specs=[pl.BlockSpec((_R, D), lambda i: (i, 0)),
                  pl.BlockSpec((D, D), lambda i: (0, 0)),
                  pl.BlockSpec((1, D), lambda i: (0, 0))],
        out_specs=[pl.BlockSpec((_R, HD), lambda i: (i, 0)),
                   pl.BlockSpec((_R, HD), lambda i: (i, 0))],
        out_shape=[jax.ShapeDtypeStruct((N, HD), _f32),
                   jax.ShapeDtypeStruct((N, HD), _f32)],
    )(x, wt, b)


def _conv_combine(m1, c1, m2, c2, xs, wlt1, wlt2, wrts, bsum):
    def body(m10, m11, c1a, c1b, m20, m21, c2a, c2b, x0, x1, w1, w2, wr, br,
             o0_ref, o1_ref):
        inv1 = 1.0 / jnp.maximum(c1a[...][:, 0:1] + c1b[...][:, 0:1], 1.0)
        inv2 = 1.0 / jnp.maximum(c2a[...][:, 0:1] + c2b[...][:, 0:1], 1.0)
        mm1 = jnp.concatenate([m10[...], m11[...]], axis=1) * inv1
        mm2 = jnp.concatenate([m20[...], m21[...]], axis=1) * inv2
        xx = jnp.concatenate([x0[...], x1[...]], axis=1)
        acc = jnp.dot(mm1, w1[...], preferred_element_type=_f32)
        acc = acc + jnp.dot(mm2, w2[...], preferred_element_type=_f32)
        acc = acc + jnp.dot(xx, wr[...], preferred_element_type=_f32)
        y = jnp.maximum(acc + br[...], 0.0)
        o0_ref[...] = y[:, :HD]
        o1_ref[...] = y[:, HD:]

    half = pl.BlockSpec((_R, HD), lambda i: (i, 0))
    cspec = pl.BlockSpec((_R, CW), lambda i: (i, 0))
    wspec = pl.BlockSpec((D, D), lambda i: (0, 0))
    return pl.pallas_call(
        body,
        grid=(N // _R,),
        in_specs=[half, half, cspec, cspec, half, half, cspec, cspec,
                  half, half,
                  wspec, wspec, wspec, pl.BlockSpec((1, D), lambda i: (0, 0))],
        out_specs=[half, half],
        out_shape=[jax.ShapeDtypeStruct((N, HD), _f32),
                   jax.ShapeDtypeStruct((N, HD), _f32)],
    )(m1[0], m1[1], c1[0], c1[1], m2[0], m2[1], c2[0], c2[1], xs[0], xs[1],
      wlt1, wlt2, wrts, bsum)


def _rule_heads(xs, w1t, b1, w2t, b2, hwt, hb):
    def body(x0, x1, w1, b1r, w2, b2r, hw, hbr, o_ref):
        x = jnp.concatenate([x0[...], x1[...]], axis=1)
        h = jnp.dot(x, w1[...], preferred_element_type=_f32) + b1r[...]
        h = jnp.maximum(h, 0.0)
        y = x + 0.1 * (jnp.dot(h, w2[...], preferred_element_type=_f32)
                       + b2r[...])
        z = jnp.dot(y, hw[...], preferred_element_type=_f32) + hbr[...]
        o_ref[...] = jax.nn.sigmoid(z)

    half = pl.BlockSpec((_R, HD), lambda i: (i, 0))
    wspec = pl.BlockSpec((D, D), lambda i: (0, 0))
    bspec = pl.BlockSpec((1, D), lambda i: (0, 0))
    return pl.pallas_call(
        body,
        grid=(N // _R,),
        in_specs=[half, half, wspec, bspec, wspec, bspec,
                  pl.BlockSpec((D, 2), lambda i: (0, 0)),
                  pl.BlockSpec((1, 2), lambda i: (0, 0))],
        out_specs=pl.BlockSpec((_R, 2), lambda i: (i, 0)),
        out_shape=jax.ShapeDtypeStruct((N, 2), _f32),
    )(xs[0], xs[1], w1t, b1, w2t, b2, hwt, hb)


def _prep_edges(e):
    pad = EPS - E // NS
    src = jnp.pad(e[0].reshape(NS, E // NS), ((0, 0), (0, pad)),
                  constant_values=0)
    dst = jnp.pad(e[1].reshape(NS, E // NS), ((0, 0), (0, pad)),
                  constant_values=N)
    return (src.reshape(NS, IR, 128), dst.reshape(NS, IR, 128))


def _prep_cnt_edges(e):
    pad = CRC * 128 - E // NW
    d = jnp.pad(e[1].reshape(NW, E // NW), ((0, 0), (0, pad)),
                constant_values=0)
    return d.reshape(NW, CRC, 128)


def kernel(x_agent, x_track, edge_observes, edge_observed_by, edge_in_fov,
           edge_in_fov_by, params):
    p = params
    seg = _build_seg_sum()
    cnt_k = _build_counts()

    def lt(q):
        return q["W"].T, q["b"][None, :]

    wat, ba = lt(p["agent_emb"])
    wtt, bt = lt(p["track_emb"])

    def conv_w(cp, et1, et2):
        wlt1 = cp[et1]["Wl"].T
        wlt2 = cp[et2]["Wl"].T
        wrts = (cp[et1]["Wr"] + cp[et2]["Wr"]).T
        bsum = (cp[et1]["bl"] + cp[et2]["bl"])[None, :]
        return wlt1, wlt2, wrts, bsum

    a1w = conv_w(p["conv1"], "observed_by", "in_fov_by")
    t1w = conv_w(p["conv1"], "observes", "in_fov")
    a2w = conv_w(p["conv2"], "observed_by", "in_fov_by")
    t2w = conv_w(p["conv2"], "observes", "in_fov")

    r = p["rule"]
    w1t, b1 = r["W1"].T, r["b1"][None, :]
    w2t, b2 = r["W2"].T, r["b2"][None, :]
    h = p["heads"]
    a_hwt = jnp.concatenate([h["agent_value"]["W"].T, h["agent_conf"]["W"].T],
                            axis=1)
    a_hb = jnp.concatenate([h["agent_value"]["b"], h["agent_conf"]["b"]])[None, :]
    t_hwt = jnp.concatenate([h["track_value"]["W"].T, h["track_conf"]["W"].T],
                            axis=1)
    t_hb = jnp.concatenate([h["track_value"]["b"], h["track_conf"]["b"]])[None, :]

    zs = jnp.zeros((RPS, HD), _f32)
    zc = jnp.zeros((RPS, CW), _f32)
    ones = jnp.ones((128, CW), _f32)

    e_ob = _prep_edges(edge_observed_by) + (zs,)
    e_ifb = _prep_edges(edge_in_fov_by) + (zs,)
    e_obs = _prep_edges(edge_observes) + (zs,)
    e_if = _prep_edges(edge_in_fov) + (zs,)

    xa = _embed(x_agent, wat, ba)
    xt = _embed(x_track, wtt, bt)

    c_ob = cnt_k(_prep_cnt_edges(edge_observed_by), ones, zc)
    c_ifb = cnt_k(_prep_cnt_edges(edge_in_fov_by), ones, zc)
    c_obs = cnt_k(_prep_cnt_edges(edge_observes), ones, zc)
    c_if = cnt_k(_prep_cnt_edges(edge_in_fov), ones, zc)

    s_ob0, s_ob1 = seg(xt[0], xt[1], *e_ob)
    s_ifb0, s_ifb1 = seg(xt[0], xt[1], *e_ifb)
    s_obs0, s_obs1 = seg(xa[0], xa[1], *e_obs)
    s_if0, s_if1 = seg(xa[0], xa[1], *e_if)
    xa1 = _conv_combine((s_ob0, s_ob1), c_ob, (s_ifb0, s_ifb1), c_ifb,
                        xa, *a1w)
    xt1 = _conv_combine((s_obs0, s_obs1), c_obs, (s_if0, s_if1), c_if,
                        xt, *t1w)

    s_ob0, s_ob1 = seg(xt1[0], xt1[1], *e_ob)
    s_ifb0, s_ifb1 = seg(xt1[0], xt1[1], *e_ifb)
    s_obs0, s_obs1 = seg(xa1[0], xa1[1], *e_obs)
    s_if0, s_if1 = seg(xa1[0], xa1[1], *e_if)
    xa2 = _conv_combine((s_ob0, s_ob1), c_ob, (s_ifb0, s_ifb1), c_ifb,
                        xa1, *a2w)
    xt2 = _conv_combine((s_obs0, s_obs1), c_obs, (s_if0, s_if1), c_if,
                        xt1, *t2w)

    za = _rule_heads(xa2, w1t, b1, w2t, b2, a_hwt, a_hb)
    zt = _rule_heads(xt2, w1t, b1, w2t, b2, t_hwt, t_hb)

    av, ac = za[:, 0:1], za[:, 1:2]
    tv, tc = zt[:, 0:1], zt[:, 1:2]
    return av, ac, tv, tc
